# baseline (speedup 1.0000x reference)
import functools

import jax
import jax.numpy as jnp
from jax import lax
from jax.experimental import pallas as pl
from jax.experimental.pallas import tpu as pltpu
from jax.experimental.pallas import tpu_sc as plsc

S, H, E, D = 2048, 1024, 8, 2048
G = H // E
SCALE = E / H
WN = SCALE * (G ** 0.5)
MIN_CAP = 4
TJ = 512
TB = 256
NB = 16
P = NB * TB
NBTOT = NB + 1
POS_NONE = 2.0 * P


def _gelu(v):
    return 0.5 * v * (1.0 + lax.erf(v * (2.0 ** -0.5)))


def _route_body(x_ref, thr_ref, perm_ref, src_ref, bexp_ref, bval_ref):
    x = x_ref[...]
    hi = lax.broadcasted_iota(jnp.int32, (H, E), 0)
    ei = lax.broadcasted_iota(jnp.int32, (H, E), 1)
    waff_t = jnp.where(hi // G == ei, jnp.float32(SCALE), jnp.float32(0.0))
    num = jnp.dot(x, waff_t)
    ssq = jnp.sum(x * x, axis=1, keepdims=True)
    den = jnp.sqrt(ssq) * WN + 1e-9
    aff = num / den
    aff_t = jnp.transpose(aff)

    mean_aff = jnp.sum(aff, keepdims=True) / (S * E)
    kf = jnp.floor(S * jax.nn.sigmoid(mean_aff - thr_ref[...]))
    kf = jnp.clip(kf, float(MIN_CAP), float(S))

    m_row = jnp.max(aff_t, axis=0, keepdims=True)
    e_iota = lax.broadcasted_iota(jnp.int32, (E, S), 0)
    top_row = jnp.min(jnp.where(aff_t == m_row, e_iota, E), axis=0,
                      keepdims=True)

    s_iota = lax.broadcasted_iota(jnp.int32, (1, S), 1)
    disp_rows = []
    for e in range(E):
        row_vals = aff_t[e:e + 1, :]
        acc = jnp.zeros((1, S), jnp.float32)
        for j in range(S // TJ):
            col_vals = aff[j * TJ:(j + 1) * TJ, e:e + 1]
            j_iota = lax.broadcasted_iota(jnp.int32, (TJ, 1), 0) + j * TJ
            gt = (col_vals > row_vals).astype(jnp.float32)
            eqlt = ((col_vals == row_vals) & (j_iota < s_iota)).astype(jnp.float32)
            acc = acc + jnp.sum(gt + eqlt, axis=0, keepdims=True)
        ecr = acc < kf
        disp_rows.append((ecr & (top_row == e)).astype(jnp.float32))
    disp_all = jnp.concatenate(disp_rows, axis=0)

    ji = lax.broadcasted_iota(jnp.int32, (S, S), 0)
    si2 = lax.broadcasted_iota(jnp.int32, (S, S), 1)
    lt = jnp.where(ji < si2, jnp.float32(1.0), jnp.float32(0.0))
    pfx = jnp.dot(disp_all, lt)

    cnt = jnp.sum(disp_all, axis=1, keepdims=True)
    alg = jnp.floor((cnt + (TB - 1)) / TB) * TB
    offs, ends = [], []
    o = jnp.zeros((1, 1), jnp.float32)
    for e in range(E):
        offs.append(o)
        o = o + alg[e:e + 1, :]
        ends.append(o)
    total = o

    base = jnp.zeros((1, S), jnp.float32)
    for e in range(E):
        base = base + disp_all[e:e + 1, :] * (offs[e] + pfx[e:e + 1, :])
    any_row = jnp.sum(disp_all, axis=0, keepdims=True)
    pos_row = jnp.where(any_row > 0, base, POS_NONE)
    src_row = jnp.where(any_row > 0, base, float(P))
    src_ref[...] = src_row.astype(jnp.int32)

    t_iota = lax.broadcasted_iota(jnp.int32, (1, S), 1).astype(jnp.float32)
    perm_tiles = []
    for pt in range(P // TJ):
        p_iota = (lax.broadcasted_iota(jnp.int32, (TJ, 1), 0) + pt * TJ
                  ).astype(jnp.float32)
        eq = (pos_row == p_iota).astype(jnp.float32)
        perm_tiles.append(jnp.sum(eq * t_iota, axis=1, keepdims=True))
    perm_ref[...] = jnp.concatenate(perm_tiles, axis=0).astype(jnp.int32)

    b_iota = lax.broadcasted_iota(jnp.int32, (32, 1), 0).astype(jnp.float32)
    bstart = b_iota * TB
    bexp = jnp.zeros((32, 1), jnp.float32)
    for e in range(E):
        bexp = bexp + (bstart >= ends[e]).astype(jnp.float32)
    bexp_ref[...] = jnp.minimum(bexp, float(E - 1)).astype(jnp.int32)
    bval = (bstart < total) & (b_iota < float(NB))
    bval_ref[...] = bval.astype(jnp.int32)


def _route(x, thr):
    return pl.pallas_call(
        _route_body,
        out_shape=(
            jax.ShapeDtypeStruct((P, 1), jnp.int32),
            jax.ShapeDtypeStruct((1, S), jnp.int32),
            jax.ShapeDtypeStruct((32, 1), jnp.int32),
            jax.ShapeDtypeStruct((32, 1), jnp.int32),
        ),
    )(x, thr)


def _const_body(b1_ref, w2_ref, b2_ref, o_ref):
    e = pl.program_id(0)
    h0 = _gelu(b1_ref[0])
    y0 = lax.dot_general(h0, w2_ref[0], (((1,), (1,)), ((), ())))
    y0 = y0 + b2_ref[0]

    @pl.when(e == 0)
    def _init():
        o_ref[...] = y0

    @pl.when(e != 0)
    def _acc():
        o_ref[...] = o_ref[...] + y0


def _const(b1, W2, b2):
    return pl.pallas_call(
        _const_body,
        grid=(E,),
        in_specs=[
            pl.BlockSpec((1, 1, D), lambda e: (e, 0, 0)),
            pl.BlockSpec((1, H, D), lambda e: (e, 0, 0)),
            pl.BlockSpec((1, 1, H), lambda e: (e, 0, 0)),
        ],
        out_specs=pl.BlockSpec((1, H), lambda e: (0, 0)),
        out_shape=jax.ShapeDtypeStruct((1, H), jnp.float32),
        compiler_params=pltpu.CompilerParams(
            dimension_semantics=("arbitrary",),
        ),
    )(b1[:, None, :], W2, b2[:, None, :])


def _make_gather(n_rows, n_chunk):
    mesh = plsc.VectorSubcoreMesh(core_axis_name="c", subcore_axis_name="s")
    NC, NS = 2, 16
    n_per = n_rows // (NC * NS)

    def body(table_hbm, idx_hbm, out_hbm, idx_v, rows_v, sem):
        wid = lax.axis_index("s") * NC + lax.axis_index("c")
        base = wid * n_per
        for c in range(n_per // n_chunk):
            off = base + c * n_chunk
            pltpu.sync_copy(idx_hbm.at[pl.ds(off, n_chunk)], idx_v)
            pltpu.async_copy(table_hbm.at[idx_v], rows_v, sem).wait()
            pltpu.sync_copy(rows_v, out_hbm.at[pl.ds(off, n_chunk)])

    return functools.partial(
        pl.kernel,
        mesh=mesh,
        out_type=jax.ShapeDtypeStruct((n_rows, H), jnp.float32),
        scratch_types=[
            pltpu.VMEM((n_chunk,), jnp.int32),
            pltpu.VMEM((n_chunk, H), jnp.float32),
            pltpu.SemaphoreType.DMA,
        ],
    )(body)


def _gather_x(table, idx):
    return _make_gather(P, 64)(table, idx)


def _gather_y(table, idx):
    return _make_gather(S, 64)(table, idx)


def _gffn_body(be_ref, bv_ref, xg_ref, w1_ref, b1_ref, w2_ref, crow_ref, o_ref):
    b = pl.program_id(0)

    @pl.when(bv_ref[b] != 0)
    def _compute():
        xb = xg_ref[...]
        h = lax.dot_general(xb, w1_ref[0], (((1,), (1,)), ((), ())))
        b1v = b1_ref[0]
        h = _gelu(h + b1v) - _gelu(b1v)
        y = lax.dot_general(h, w2_ref[0], (((1,), (1,)), ((), ())))
        o_ref[...] = y + crow_ref[...]

    @pl.when(b == NB)
    def _trash():
        o_ref[...] = jnp.broadcast_to(crow_ref[...], (TB, H))


def _gffn(bexp, bval, xg, W1, b1, W2, crow):
    grid_spec = pltpu.PrefetchScalarGridSpec(
        num_scalar_prefetch=2,
        grid=(NBTOT,),
        in_specs=[
            pl.BlockSpec((TB, H), lambda b, be, bv: (jnp.minimum(b, NB - 1), 0)),
            pl.BlockSpec((1, D, H), lambda b, be, bv: (be[b], 0, 0)),
            pl.BlockSpec((1, 1, D), lambda b, be, bv: (be[b], 0, 0)),
            pl.BlockSpec((1, H, D), lambda b, be, bv: (be[b], 0, 0)),
            pl.BlockSpec((1, H), lambda b, be, bv: (0, 0)),
        ],
        out_specs=pl.BlockSpec((TB, H), lambda b, be, bv: (b, 0)),
    )
    return pl.pallas_call(
        _gffn_body,
        grid_spec=grid_spec,
        out_shape=jax.ShapeDtypeStruct((NBTOT * TB, H), jnp.float32),
        compiler_params=pltpu.CompilerParams(
            dimension_semantics=("arbitrary",),
        ),
    )(bexp, bval, xg, W1, b1[:, None, :], W2, crow)


def kernel(inputs, W1, b1, W2, b2, affinity_threshold):
    x = inputs[0]
    thr = jnp.reshape(affinity_threshold, (1, 1)).astype(jnp.float32)
    perm, src, bexp, bval = _route(x, thr)
    crow = _const(b1, W2, b2)
    xg = _gather_x(x, perm.reshape(P))
    yg = _gffn(bexp.reshape(32), bval.reshape(32), xg, W1, b1, W2, crow)
    out = _gather_y(yg, src.reshape(S))
    return out[None]


# speedup vs baseline: 1.6111x; 1.6111x over previous
import functools

import jax
import jax.numpy as jnp
from jax import lax
from jax.experimental import pallas as pl
from jax.experimental.pallas import tpu as pltpu
from jax.experimental.pallas import tpu_sc as plsc

S, H, E, D = 2048, 1024, 8, 2048
G = H // E
SCALE = E / H
WN = SCALE * (G ** 0.5)
MIN_CAP = 4
TJ = 512
TB = 256
NB = 16
P = NB * TB
NBTOT = NB + 1
POS_NONE = 2.0 * P


def _gelu(v):
    return 0.5 * v * (1.0 + lax.erf(v * (2.0 ** -0.5)))


def _route_body(x_ref, thr_ref, perm_ref, src_ref, bexp_ref, bval_ref):
    x = x_ref[...]
    hi = lax.broadcasted_iota(jnp.int32, (H, E), 0)
    ei = lax.broadcasted_iota(jnp.int32, (H, E), 1)
    waff_t = jnp.where(hi // G == ei, jnp.float32(SCALE), jnp.float32(0.0))
    num = jnp.dot(x, waff_t)
    ssq = jnp.sum(x * x, axis=1, keepdims=True)
    den = jnp.sqrt(ssq) * WN + 1e-9
    aff = num / den
    aff_t = jnp.transpose(aff)

    mean_aff = jnp.sum(aff, keepdims=True) / (S * E)
    kf = jnp.floor(S * jax.nn.sigmoid(mean_aff - thr_ref[...]))
    kf = jnp.clip(kf, float(MIN_CAP), float(S))

    m_row = jnp.max(aff_t, axis=0, keepdims=True)
    e_iota = lax.broadcasted_iota(jnp.int32, (E, S), 0)
    top_row = jnp.min(jnp.where(aff_t == m_row, e_iota, E), axis=0,
                      keepdims=True)

    s_iota = lax.broadcasted_iota(jnp.int32, (1, S), 1)
    disp_rows = []
    for e in range(E):
        row_vals = aff_t[e:e + 1, :]
        acc = jnp.zeros((1, S), jnp.float32)
        for j in range(S // TJ):
            col_vals = aff[j * TJ:(j + 1) * TJ, e:e + 1]
            j_iota = lax.broadcasted_iota(jnp.int32, (TJ, 1), 0) + j * TJ
            gt = (col_vals > row_vals).astype(jnp.float32)
            eqlt = ((col_vals == row_vals) & (j_iota < s_iota)).astype(jnp.float32)
            acc = acc + jnp.sum(gt + eqlt, axis=0, keepdims=True)
        ecr = acc < kf
        disp_rows.append((ecr & (top_row == e)).astype(jnp.float32))
    disp_all = jnp.concatenate(disp_rows, axis=0)

    ji = lax.broadcasted_iota(jnp.int32, (S, S), 0)
    si2 = lax.broadcasted_iota(jnp.int32, (S, S), 1)
    lt = jnp.where(ji < si2, jnp.float32(1.0), jnp.float32(0.0))
    pfx = jnp.dot(disp_all, lt)

    cnt = jnp.sum(disp_all, axis=1, keepdims=True)
    alg = jnp.floor((cnt + (TB - 1)) / TB) * TB
    offs, ends = [], []
    o = jnp.zeros((1, 1), jnp.float32)
    for e in range(E):
        offs.append(o)
        o = o + alg[e:e + 1, :]
        ends.append(o)
    total = o

    base = jnp.zeros((1, S), jnp.float32)
    for e in range(E):
        base = base + disp_all[e:e + 1, :] * (offs[e] + pfx[e:e + 1, :])
    any_row = jnp.sum(disp_all, axis=0, keepdims=True)
    pos_row = jnp.where(any_row > 0, base, POS_NONE)
    src_row = jnp.where(any_row > 0, base, float(P))
    src_ref[...] = src_row.astype(jnp.int32)

    t_iota = lax.broadcasted_iota(jnp.int32, (1, S), 1).astype(jnp.float32)
    perm_tiles = []
    for pt in range(P // TJ):
        p_iota = (lax.broadcasted_iota(jnp.int32, (TJ, 1), 0) + pt * TJ
                  ).astype(jnp.float32)
        eq = (pos_row == p_iota).astype(jnp.float32)
        val = jnp.sum(eq * t_iota, axis=1, keepdims=True)
        nonempty = jnp.sum(eq, axis=1, keepdims=True) > 0
        spread = p_iota - jnp.where(p_iota >= float(S), float(S), 0.0)
        perm_tiles.append(jnp.where(nonempty, val, spread))
    perm_ref[...] = jnp.concatenate(perm_tiles, axis=0).astype(jnp.int32)

    b_iota = lax.broadcasted_iota(jnp.int32, (32, 1), 0).astype(jnp.float32)
    bstart = b_iota * TB
    bexp = jnp.zeros((32, 1), jnp.float32)
    for e in range(E):
        bexp = bexp + (bstart >= ends[e]).astype(jnp.float32)
    bexp_ref[...] = jnp.minimum(bexp, float(E - 1)).astype(jnp.int32)
    bval = (bstart < total) & (b_iota < float(NB))
    bval_ref[...] = bval.astype(jnp.int32)


def _route(x, thr):
    return pl.pallas_call(
        _route_body,
        out_shape=(
            jax.ShapeDtypeStruct((P, 1), jnp.int32),
            jax.ShapeDtypeStruct((1, S), jnp.int32),
            jax.ShapeDtypeStruct((32, 1), jnp.int32),
            jax.ShapeDtypeStruct((32, 1), jnp.int32),
        ),
    )(x, thr)


def _make_gather(n_rows, n_chunk):
    mesh = plsc.VectorSubcoreMesh(core_axis_name="c", subcore_axis_name="s")
    NC, NS = 2, 16
    n_per = n_rows // (NC * NS)

    def body(table_hbm, idx_hbm, out_hbm, idx_v, rows_v, sem):
        wid = lax.axis_index("s") * NC + lax.axis_index("c")
        base = wid * n_per
        for c in range(n_per // n_chunk):
            off = base + c * n_chunk
            pltpu.sync_copy(idx_hbm.at[pl.ds(off, n_chunk)], idx_v)
            pltpu.async_copy(table_hbm.at[idx_v], rows_v, sem).wait()
            pltpu.sync_copy(rows_v, out_hbm.at[pl.ds(off, n_chunk)])

    return functools.partial(
        pl.kernel,
        mesh=mesh,
        out_type=jax.ShapeDtypeStruct((n_rows, H), jnp.float32),
        scratch_types=[
            pltpu.VMEM((n_chunk,), jnp.int32),
            pltpu.VMEM((n_chunk, H), jnp.float32),
            pltpu.SemaphoreType.DMA,
        ],
    )(body)


def _gather_x(table, idx):
    return _make_gather(P, 64)(table, idx)


def _gather_y(table, idx):
    return _make_gather(S, 64)(table, idx)


def _gffn_body(be_ref, bv_ref, xg_ref, w1_ref, b1_ref, w2_ref, b2_ref, o_ref,
               crow_scr):
    i = pl.program_id(0)
    b = i - E
    bc = jnp.maximum(b, 0)

    @pl.when(i < E)
    def _const_step():
        h0 = _gelu(b1_ref[0])
        y0 = lax.dot_general(h0, w2_ref[0], (((1,), (1,)), ((), ())))
        y0 = y0 + b2_ref[0]

        @pl.when(i == 0)
        def _init():
            crow_scr[...] = y0

        @pl.when(i != 0)
        def _acc():
            crow_scr[...] = crow_scr[...] + y0

    @pl.when((i >= E) & (bv_ref[bc] != 0))
    def _compute():
        xb = xg_ref[...]
        h = lax.dot_general(xb, w1_ref[0], (((1,), (1,)), ((), ())))
        b1v = b1_ref[0]
        h = _gelu(h + b1v) - _gelu(b1v)
        y = lax.dot_general(h, w2_ref[0], (((1,), (1,)), ((), ())))
        o_ref[...] = y + crow_scr[...]

    @pl.when(b == NB)
    def _trash():
        o_ref[...] = jnp.broadcast_to(crow_scr[...], (TB, H))


def _gffn(bexp, bval, xg, W1, b1, W2, b2):
    def _eidx(i, be):
        return jnp.where(i < E, i, be[jnp.maximum(i - E, 0)])

    grid_spec = pltpu.PrefetchScalarGridSpec(
        num_scalar_prefetch=2,
        grid=(E + NBTOT,),
        in_specs=[
            pl.BlockSpec(
                (TB, H),
                lambda i, be, bv: (jnp.minimum(jnp.maximum(i - E, 0), NB - 1), 0)),
            pl.BlockSpec(
                (1, D, H),
                lambda i, be, bv: (be[jnp.maximum(i - E, 0)], 0, 0)),
            pl.BlockSpec((1, 1, D), lambda i, be, bv: (_eidx(i, be), 0, 0)),
            pl.BlockSpec((1, H, D), lambda i, be, bv: (_eidx(i, be), 0, 0)),
            pl.BlockSpec((1, 1, H), lambda i, be, bv: (_eidx(i, be), 0, 0)),
        ],
        out_specs=pl.BlockSpec(
            (TB, H),
            lambda i, be, bv: (jnp.minimum(jnp.maximum(i - E, 0), NB), 0)),
        scratch_shapes=[pltpu.VMEM((1, H), jnp.float32)],
    )
    return pl.pallas_call(
        _gffn_body,
        grid_spec=grid_spec,
        out_shape=jax.ShapeDtypeStruct((NBTOT * TB, H), jnp.float32),
        compiler_params=pltpu.CompilerParams(
            dimension_semantics=("arbitrary",),
        ),
    )(bexp, bval, xg, W1, b1[:, None, :], W2, b2[:, None, :])


def kernel(inputs, W1, b1, W2, b2, affinity_threshold):
    x = inputs[0]
    thr = jnp.reshape(affinity_threshold, (1, 1)).astype(jnp.float32)
    perm, src, bexp, bval = _route(x, thr)
    xg = _gather_x(x, perm.reshape(P))
    yg = _gffn(bexp.reshape(32), bval.reshape(32), xg, W1, b1, W2, b2)
    out = _gather_y(yg, src.reshape(S))
    return out[None]


# speedup vs baseline: 1.8344x; 1.1386x over previous
import functools

import jax
import jax.numpy as jnp
from jax import lax
from jax.experimental import pallas as pl
from jax.experimental.pallas import tpu as pltpu
from jax.experimental.pallas import tpu_sc as plsc

S, H, E, D = 2048, 1024, 8, 2048
G = H // E
SCALE = E / H
WN = SCALE * (G ** 0.5)
MIN_CAP = 4
TJ = 512
TB = 256
NB = 16
P = NB * TB
NBTOT = NB + 1
POS_NONE = 2.0 * P


def _gelu(v):
    return 0.5 * v * (1.0 + lax.erf(v * (2.0 ** -0.5)))


def _route_body(x_ref, thr_ref, perm_ref, src_ref, bexp_ref, bval_ref):
    x = x_ref[...]
    hi = lax.broadcasted_iota(jnp.int32, (H, E), 0)
    ei = lax.broadcasted_iota(jnp.int32, (H, E), 1)
    waff_t = jnp.where(hi // G == ei, jnp.float32(SCALE), jnp.float32(0.0))
    num = jnp.dot(x, waff_t)
    ssq = jnp.sum(x * x, axis=1, keepdims=True)
    den = jnp.sqrt(ssq) * WN + 1e-9
    aff = num / den
    aff_t = jnp.transpose(aff)

    mean_aff = jnp.sum(aff, keepdims=True) / (S * E)
    kf = jnp.floor(S * jax.nn.sigmoid(mean_aff - thr_ref[...]))
    kf = jnp.clip(kf, float(MIN_CAP), float(S))

    m_row = jnp.max(aff_t, axis=0, keepdims=True)
    e_iota = lax.broadcasted_iota(jnp.int32, (E, S), 0)
    top_row = jnp.min(jnp.where(aff_t == m_row, e_iota, E), axis=0,
                      keepdims=True)

    ji = lax.broadcasted_iota(jnp.int32, (S, S), 0)
    si2 = lax.broadcasted_iota(jnp.int32, (S, S), 1)
    lt = jnp.where(ji < si2, jnp.float32(1.0), jnp.float32(0.0))

    bits = lax.bitcast_convert_type(aff_t + 0.0, jnp.int32)
    keys = bits ^ (jnp.right_shift(bits, 31) & jnp.int32(0x7FFFFFFF))
    ki = kf.astype(jnp.int32)
    lo = jnp.full((E, 1), -(2 ** 30), jnp.int32)
    hi = jnp.full((E, 1), 2 ** 30, jnp.int32)
    for _ in range(31):
        mid = jnp.right_shift(lo + hi, 1)
        cnt = jnp.sum((keys > mid).astype(jnp.int32), axis=1, keepdims=True)
        p = cnt < ki
        hi = jnp.where(p, mid, hi)
        lo = jnp.where(p, lo, mid)
    tau = hi
    gt = keys > tau
    cnt_gt = jnp.sum(gt.astype(jnp.int32), axis=1, keepdims=True)
    eqm = keys == tau
    eq_pfx = jnp.dot(eqm.astype(jnp.float32), lt)
    ecr_all = gt | (eqm & (eq_pfx < (ki - cnt_gt).astype(jnp.float32)))
    disp_all = (ecr_all & (e_iota == top_row)).astype(jnp.float32)

    pfx = jnp.dot(disp_all, lt)

    cnt = jnp.sum(disp_all, axis=1, keepdims=True)
    alg = jnp.floor((cnt + (TB - 1)) / TB) * TB
    offs, ends = [], []
    o = jnp.zeros((1, 1), jnp.float32)
    for e in range(E):
        offs.append(o)
        o = o + alg[e:e + 1, :]
        ends.append(o)
    total = o

    base = jnp.zeros((1, S), jnp.float32)
    for e in range(E):
        base = base + disp_all[e:e + 1, :] * (offs[e] + pfx[e:e + 1, :])
    any_row = jnp.sum(disp_all, axis=0, keepdims=True)
    pos_row = jnp.where(any_row > 0, base, POS_NONE)
    src_row = jnp.where(any_row > 0, base, float(P))
    src_ref[...] = src_row.astype(jnp.int32)

    t_iota = lax.broadcasted_iota(jnp.int32, (1, S), 1).astype(jnp.float32)
    perm_tiles = []
    for pt in range(P // TJ):
        p_iota = (lax.broadcasted_iota(jnp.int32, (TJ, 1), 0) + pt * TJ
                  ).astype(jnp.float32)
        eq = (pos_row == p_iota).astype(jnp.float32)
        val = jnp.sum(eq * t_iota, axis=1, keepdims=True)
        nonempty = jnp.sum(eq, axis=1, keepdims=True) > 0
        spread = p_iota - jnp.where(p_iota >= float(S), float(S), 0.0)
        perm_tiles.append(jnp.where(nonempty, val, spread))
    perm_ref[...] = jnp.concatenate(perm_tiles, axis=0).astype(jnp.int32)

    b_iota = lax.broadcasted_iota(jnp.int32, (32, 1), 0).astype(jnp.float32)
    bstart = b_iota * TB
    bexp = jnp.zeros((32, 1), jnp.float32)
    for e in range(E):
        bexp = bexp + (bstart >= ends[e]).astype(jnp.float32)
    bexp_ref[...] = jnp.minimum(bexp, float(E - 1)).astype(jnp.int32)
    bval = (bstart < total) & (b_iota < float(NB))
    bval_ref[...] = bval.astype(jnp.int32)


def _route(x, thr):
    return pl.pallas_call(
        _route_body,
        out_shape=(
            jax.ShapeDtypeStruct((P, 1), jnp.int32),
            jax.ShapeDtypeStruct((1, S), jnp.int32),
            jax.ShapeDtypeStruct((32, 1), jnp.int32),
            jax.ShapeDtypeStruct((32, 1), jnp.int32),
        ),
    )(x, thr)


def _make_gather(n_rows, n_chunk):
    mesh = plsc.VectorSubcoreMesh(core_axis_name="c", subcore_axis_name="s")
    NC, NS = 2, 16
    n_per = n_rows // (NC * NS)

    def body(table_hbm, idx_hbm, out_hbm, idx_v, rows_v, sem):
        wid = lax.axis_index("s") * NC + lax.axis_index("c")
        base = wid * n_per
        for c in range(n_per // n_chunk):
            off = base + c * n_chunk
            pltpu.sync_copy(idx_hbm.at[pl.ds(off, n_chunk)], idx_v)
            pltpu.async_copy(table_hbm.at[idx_v], rows_v, sem).wait()
            pltpu.sync_copy(rows_v, out_hbm.at[pl.ds(off, n_chunk)])

    return functools.partial(
        pl.kernel,
        mesh=mesh,
        out_type=jax.ShapeDtypeStruct((n_rows, H), jnp.float32),
        scratch_types=[
            pltpu.VMEM((n_chunk,), jnp.int32),
            pltpu.VMEM((n_chunk, H), jnp.float32),
            pltpu.SemaphoreType.DMA,
        ],
    )(body)


def _gather_x(table, idx):
    return _make_gather(P, 64)(table, idx)


def _gather_y(table, idx):
    return _make_gather(S, 64)(table, idx)


def _gffn_body(be_ref, bv_ref, xg_ref, w1_ref, b1_ref, w2_ref, b2_ref, o_ref,
               crow_scr):
    i = pl.program_id(0)
    b = i - E
    bc = jnp.maximum(b, 0)

    @pl.when(i < E)
    def _const_step():
        h0 = _gelu(b1_ref[0])
        y0 = lax.dot_general(h0, w2_ref[0], (((1,), (1,)), ((), ())))
        y0 = y0 + b2_ref[0]

        @pl.when(i == 0)
        def _init():
            crow_scr[...] = y0

        @pl.when(i != 0)
        def _acc():
            crow_scr[...] = crow_scr[...] + y0

    @pl.when((i >= E) & (bv_ref[bc] != 0))
    def _compute():
        xb = xg_ref[...]
        h = lax.dot_general(xb, w1_ref[0], (((1,), (1,)), ((), ())))
        b1v = b1_ref[0]
        h = _gelu(h + b1v) - _gelu(b1v)
        y = lax.dot_general(h, w2_ref[0], (((1,), (1,)), ((), ())))
        o_ref[...] = y + crow_scr[...]

    @pl.when(b == NB)
    def _trash():
        o_ref[...] = jnp.broadcast_to(crow_scr[...], (TB, H))


def _gffn(bexp, bval, xg, W1, b1, W2, b2):
    def _eidx(i, be):
        return jnp.where(i < E, i, be[jnp.maximum(i - E, 0)])

    grid_spec = pltpu.PrefetchScalarGridSpec(
        num_scalar_prefetch=2,
        grid=(E + NBTOT,),
        in_specs=[
            pl.BlockSpec(
                (TB, H),
                lambda i, be, bv: (jnp.minimum(jnp.maximum(i - E, 0), NB - 1), 0)),
            pl.BlockSpec(
                (1, D, H),
                lambda i, be, bv: (be[jnp.maximum(i - E, 0)], 0, 0)),
            pl.BlockSpec((1, 1, D), lambda i, be, bv: (_eidx(i, be), 0, 0)),
            pl.BlockSpec((1, H, D), lambda i, be, bv: (_eidx(i, be), 0, 0)),
            pl.BlockSpec((1, 1, H), lambda i, be, bv: (_eidx(i, be), 0, 0)),
        ],
        out_specs=pl.BlockSpec(
            (TB, H),
            lambda i, be, bv: (jnp.minimum(jnp.maximum(i - E, 0), NB), 0)),
        scratch_shapes=[pltpu.VMEM((1, H), jnp.float32)],
    )
    return pl.pallas_call(
        _gffn_body,
        grid_spec=grid_spec,
        out_shape=jax.ShapeDtypeStruct((NBTOT * TB, H), jnp.float32),
        compiler_params=pltpu.CompilerParams(
            dimension_semantics=("arbitrary",),
        ),
    )(bexp, bval, xg, W1, b1[:, None, :], W2, b2[:, None, :])


def kernel(inputs, W1, b1, W2, b2, affinity_threshold):
    x = inputs[0]
    thr = jnp.reshape(affinity_threshold, (1, 1)).astype(jnp.float32)
    perm, src, bexp, bval = _route(x, thr)
    xg = _gather_x(x, perm.reshape(P))
    yg = _gffn(bexp.reshape(32), bval.reshape(32), xg, W1, b1, W2, b2)
    out = _gather_y(yg, src.reshape(S))
    return out[None]


# speedup vs baseline: 1.9231x; 1.0483x over previous
import functools

import jax
import jax.numpy as jnp
from jax import lax
from jax.experimental import pallas as pl
from jax.experimental.pallas import tpu as pltpu
from jax.experimental.pallas import tpu_sc as plsc

S, H, E, D = 2048, 1024, 8, 2048
G = H // E
SCALE = E / H
WN = SCALE * (G ** 0.5)
MIN_CAP = 4
TJ = 512
TB = 256
NB = 16
P = NB * TB
NBTOT = NB + 1
POS_NONE = 2.0 * P


def _gelu(v):
    return 0.5 * v * (1.0 + lax.erf(v * (2.0 ** -0.5)))


def _route_body(x_ref, thr_ref, perm_ref, src_ref, bexp_ref, bval_ref,
                first_ref, absent_ref):
    x = x_ref[...]
    hi = lax.broadcasted_iota(jnp.int32, (H, E), 0)
    ei = lax.broadcasted_iota(jnp.int32, (H, E), 1)
    waff_t = jnp.where(hi // G == ei, jnp.float32(SCALE), jnp.float32(0.0))
    num = jnp.dot(x, waff_t)
    ssq = jnp.sum(x * x, axis=1, keepdims=True)
    den = jnp.sqrt(ssq) * WN + 1e-9
    aff = num / den
    aff_t = jnp.transpose(aff)

    mean_aff = jnp.sum(aff, keepdims=True) / (S * E)
    kf = jnp.floor(S * jax.nn.sigmoid(mean_aff - thr_ref[...]))
    kf = jnp.clip(kf, float(MIN_CAP), float(S))

    m_row = jnp.max(aff_t, axis=0, keepdims=True)
    e_iota = lax.broadcasted_iota(jnp.int32, (E, S), 0)
    top_row = jnp.min(jnp.where(aff_t == m_row, e_iota, E), axis=0,
                      keepdims=True)

    ji = lax.broadcasted_iota(jnp.int32, (S, S), 0)
    si2 = lax.broadcasted_iota(jnp.int32, (S, S), 1)
    lt = jnp.where(ji < si2, jnp.float32(1.0), jnp.float32(0.0))

    bits = lax.bitcast_convert_type(aff_t + 0.0, jnp.int32)
    keys = bits ^ (jnp.right_shift(bits, 31) & jnp.int32(0x7FFFFFFF))
    ki = kf.astype(jnp.int32)
    lo = jnp.full((E, 1), -(2 ** 30), jnp.int32)
    hi = jnp.full((E, 1), 2 ** 30, jnp.int32)
    for _ in range(31):
        mid = jnp.right_shift(lo + hi, 1)
        cnt = jnp.sum((keys > mid).astype(jnp.int32), axis=1, keepdims=True)
        p = cnt < ki
        hi = jnp.where(p, mid, hi)
        lo = jnp.where(p, lo, mid)
    tau = hi
    gt = keys > tau
    cnt_gt = jnp.sum(gt.astype(jnp.int32), axis=1, keepdims=True)
    eqm = keys == tau
    eq_pfx = jnp.dot(eqm.astype(jnp.float32), lt)
    ecr_all = gt | (eqm & (eq_pfx < (ki - cnt_gt).astype(jnp.float32)))
    disp_all = (ecr_all & (e_iota == top_row)).astype(jnp.float32)

    pfx = jnp.dot(disp_all, lt)

    cnt = jnp.sum(disp_all, axis=1, keepdims=True)
    alg = jnp.floor((cnt + (TB - 1)) / TB) * TB
    offs, ends = [], []
    o = jnp.zeros((1, 1), jnp.float32)
    for e in range(E):
        offs.append(o)
        o = o + alg[e:e + 1, :]
        ends.append(o)
    total = o

    base = jnp.zeros((1, S), jnp.float32)
    for e in range(E):
        base = base + disp_all[e:e + 1, :] * (offs[e] + pfx[e:e + 1, :])
    any_row = jnp.sum(disp_all, axis=0, keepdims=True)
    pos_row = jnp.where(any_row > 0, base, POS_NONE)
    src_row = jnp.where(any_row > 0, base, float(P))
    src_ref[...] = src_row.astype(jnp.int32)

    t_iota = lax.broadcasted_iota(jnp.int32, (1, S), 1).astype(jnp.float32)
    perm_tiles = []
    for pt in range(P // TJ):
        p_iota = (lax.broadcasted_iota(jnp.int32, (TJ, 1), 0) + pt * TJ
                  ).astype(jnp.float32)
        eq = (pos_row == p_iota).astype(jnp.float32)
        val = jnp.sum(eq * t_iota, axis=1, keepdims=True)
        nonempty = jnp.sum(eq, axis=1, keepdims=True) > 0
        spread = p_iota - jnp.where(p_iota >= float(S), float(S), 0.0)
        perm_tiles.append(jnp.where(nonempty, val, spread))
    perm_ref[...] = jnp.concatenate(perm_tiles, axis=0).astype(jnp.int32)

    b_iota = lax.broadcasted_iota(jnp.int32, (32, 1), 0).astype(jnp.float32)
    bstart = b_iota * TB
    bexp = jnp.zeros((32, 1), jnp.float32)
    for e in range(E):
        bexp = bexp + (bstart >= ends[e]).astype(jnp.float32)
    bexp = jnp.minimum(bexp, float(E - 1))
    bexp_ref[...] = bexp.astype(jnp.int32)
    bval = (bstart < total) & (b_iota < float(NB))
    bval_ref[...] = bval.astype(jnp.int32)
    prev = jnp.concatenate([jnp.full((1, 1), -1.0), bexp[:31, :]], axis=0)
    first_ref[...] = (bval & (bexp != prev)).astype(jnp.int32)
    absent8 = (cnt == 0).astype(jnp.int32)
    absent_ref[...] = jnp.concatenate(
        [absent8, jnp.zeros((32 - E, 1), jnp.int32)], axis=0)


def _route(x, thr):
    return pl.pallas_call(
        _route_body,
        out_shape=(
            jax.ShapeDtypeStruct((P, 1), jnp.int32),
            jax.ShapeDtypeStruct((1, S), jnp.int32),
            jax.ShapeDtypeStruct((32, 1), jnp.int32),
            jax.ShapeDtypeStruct((32, 1), jnp.int32),
            jax.ShapeDtypeStruct((32, 1), jnp.int32),
            jax.ShapeDtypeStruct((32, 1), jnp.int32),
        ),
    )(x, thr)


def _make_gather(n_rows, n_chunk):
    mesh = plsc.VectorSubcoreMesh(core_axis_name="c", subcore_axis_name="s")
    NC, NS = 2, 16
    n_per = n_rows // (NC * NS)

    def body(table_hbm, idx_hbm, out_hbm, idx_v, rows_v, sem):
        wid = lax.axis_index("s") * NC + lax.axis_index("c")
        base = wid * n_per
        for c in range(n_per // n_chunk):
            off = base + c * n_chunk
            pltpu.sync_copy(idx_hbm.at[pl.ds(off, n_chunk)], idx_v)
            pltpu.async_copy(table_hbm.at[idx_v], rows_v, sem).wait()
            pltpu.sync_copy(rows_v, out_hbm.at[pl.ds(off, n_chunk)])

    return functools.partial(
        pl.kernel,
        mesh=mesh,
        out_type=jax.ShapeDtypeStruct((n_rows, H), jnp.float32),
        scratch_types=[
            pltpu.VMEM((n_chunk,), jnp.int32),
            pltpu.VMEM((n_chunk, H), jnp.float32),
            pltpu.SemaphoreType.DMA,
        ],
    )(body)


def _gather_x(table, idx):
    return _make_gather(P, 64)(table, idx)


def _gather_y(table, idx):
    return _make_gather(S, 64)(table, idx)


def _gffn_body(be_ref, bv_ref, ab_ref, fr_ref, xg_ref, w1_ref, b1_ref, w2_ref,
               b2_ref, o_ref, crow_ref, crow_scr):
    i = pl.program_id(0)
    b = i - E
    bc = jnp.maximum(b, 0)

    @pl.when(i == 0)
    def _init():
        crow_scr[...] = jnp.zeros((1, H), jnp.float32)

    @pl.when(((i < E) & (ab_ref[jnp.minimum(i, E - 1)] != 0))
             | ((i >= E) & (fr_ref[bc] != 0)))
    def _const_step():
        h0 = _gelu(b1_ref[0])
        y0 = lax.dot_general(h0, w2_ref[0], (((1,), (1,)), ((), ())))
        crow_scr[...] = crow_scr[...] + y0 + b2_ref[0]

    @pl.when((i >= E) & (bv_ref[bc] != 0))
    def _compute():
        xb = xg_ref[...]
        h = lax.dot_general(xb, w1_ref[0], (((1,), (1,)), ((), ())))
        b1v = b1_ref[0]
        h = _gelu(h + b1v) - _gelu(b1v)
        y = lax.dot_general(h, w2_ref[0], (((1,), (1,)), ((), ())))
        o_ref[...] = y

    @pl.when(b == NB)
    def _trash():
        o_ref[...] = jnp.zeros((TB, H), jnp.float32)

    crow_ref[...] = crow_scr[...]


def _gffn(bexp, bval, absent, first, xg, W1, b1, W2, b2):
    def _eidx(i, be, ab):
        pro = jnp.where(ab[jnp.minimum(i, E - 1)] != 0, i, be[0])
        return jnp.where(i < E, pro, be[jnp.maximum(i - E, 0)])

    grid_spec = pltpu.PrefetchScalarGridSpec(
        num_scalar_prefetch=4,
        grid=(E + NBTOT,),
        in_specs=[
            pl.BlockSpec(
                (TB, H),
                lambda i, be, bv, ab, fr: (
                    jnp.minimum(jnp.maximum(i - E, 0), NB - 1), 0)),
            pl.BlockSpec(
                (1, D, H),
                lambda i, be, bv, ab, fr: (be[jnp.maximum(i - E, 0)], 0, 0)),
            pl.BlockSpec(
                (1, 1, D),
                lambda i, be, bv, ab, fr: (_eidx(i, be, ab), 0, 0)),
            pl.BlockSpec(
                (1, H, D),
                lambda i, be, bv, ab, fr: (_eidx(i, be, ab), 0, 0)),
            pl.BlockSpec(
                (1, 1, H),
                lambda i, be, bv, ab, fr: (_eidx(i, be, ab), 0, 0)),
        ],
        out_specs=(
            pl.BlockSpec(
                (TB, H),
                lambda i, be, bv, ab, fr: (
                    jnp.minimum(jnp.maximum(i - E, 0), NB), 0)),
            pl.BlockSpec((1, H), lambda i, be, bv, ab, fr: (0, 0)),
        ),
        scratch_shapes=[pltpu.VMEM((1, H), jnp.float32)],
    )
    return pl.pallas_call(
        _gffn_body,
        grid_spec=grid_spec,
        out_shape=(
            jax.ShapeDtypeStruct((NBTOT * TB, H), jnp.float32),
            jax.ShapeDtypeStruct((1, H), jnp.float32),
        ),
        compiler_params=pltpu.CompilerParams(
            dimension_semantics=("arbitrary",),
        ),
    )(bexp, bval, absent, first, xg, W1, b1[:, None, :], W2, b2[:, None, :])


def _addc_body(a_ref, c_ref, o_ref):
    o_ref[...] = a_ref[...] + c_ref[...]


def _addc(a, c):
    return pl.pallas_call(
        _addc_body,
        out_shape=jax.ShapeDtypeStruct((S, H), jnp.float32),
    )(a, c)


def kernel(inputs, W1, b1, W2, b2, affinity_threshold):
    x = inputs[0]
    thr = jnp.reshape(affinity_threshold, (1, 1)).astype(jnp.float32)
    perm, src, bexp, bval, first, absent = _route(x, thr)
    xg = _gather_x(x, perm.reshape(P))
    yg, crow = _gffn(bexp.reshape(32), bval.reshape(32), absent.reshape(32),
                     first.reshape(32), xg, W1, b1, W2, b2)
    out = _gather_y(yg, src.reshape(S))
    return _addc(out, crow)[None]


# speedup vs baseline: 2.0056x; 1.0429x over previous
import functools

import jax
import jax.numpy as jnp
from jax import lax
from jax.experimental import pallas as pl
from jax.experimental.pallas import tpu as pltpu
from jax.experimental.pallas import tpu_sc as plsc

S, H, E, D = 2048, 1024, 8, 2048
G = H // E
SCALE = E / H
WN = SCALE * (G ** 0.5)
MIN_CAP = 4
TJ = 512
TB = 256
NB = 16
P = NB * TB
NBTOT = NB + 1
POS_NONE = 2.0 * P


def _gelu(v):
    return 0.5 * v * (1.0 + lax.erf(v * (2.0 ** -0.5)))


def _route_body(x_ref, thr_ref, perm_ref, src_ref, bexp_ref, bval_ref,
                first_ref):
    x = x_ref[...]
    hi = lax.broadcasted_iota(jnp.int32, (H, E), 0)
    ei = lax.broadcasted_iota(jnp.int32, (H, E), 1)
    waff_t = jnp.where(hi // G == ei, jnp.float32(SCALE), jnp.float32(0.0))
    num = jnp.dot(x, waff_t)
    ssq = jnp.sum(x * x, axis=1, keepdims=True)
    den = jnp.sqrt(ssq) * WN + 1e-9
    aff = num / den
    aff_t = jnp.transpose(aff)

    mean_aff = jnp.sum(aff, keepdims=True) / (S * E)
    kf = jnp.floor(S * jax.nn.sigmoid(mean_aff - thr_ref[...]))
    kf = jnp.clip(kf, float(MIN_CAP), float(S))

    m_row = jnp.max(aff_t, axis=0, keepdims=True)
    e_iota = lax.broadcasted_iota(jnp.int32, (E, S), 0)
    top_row = jnp.min(jnp.where(aff_t == m_row, e_iota, E), axis=0,
                      keepdims=True)

    ji = lax.broadcasted_iota(jnp.int32, (S, S), 0)
    si2 = lax.broadcasted_iota(jnp.int32, (S, S), 1)
    lt = jnp.where(ji < si2, jnp.float32(1.0), jnp.float32(0.0))

    bits = lax.bitcast_convert_type(aff_t + 0.0, jnp.int32)
    keys = bits ^ (jnp.right_shift(bits, 31) & jnp.int32(0x7FFFFFFF))
    ki = kf.astype(jnp.int32)
    lo = jnp.full((E, 1), -(2 ** 30), jnp.int32)
    hi = jnp.full((E, 1), 2 ** 30, jnp.int32)
    for _ in range(31):
        mid = jnp.right_shift(lo + hi, 1)
        cnt = jnp.sum((keys > mid).astype(jnp.int32), axis=1, keepdims=True)
        p = cnt < ki
        hi = jnp.where(p, mid, hi)
        lo = jnp.where(p, lo, mid)
    tau = hi
    gt = keys > tau
    cnt_gt = jnp.sum(gt.astype(jnp.int32), axis=1, keepdims=True)
    eqm = keys == tau
    eq_pfx = jnp.dot(eqm.astype(jnp.float32), lt)
    ecr_all = gt | (eqm & (eq_pfx < (ki - cnt_gt).astype(jnp.float32)))
    disp_all = (ecr_all & (e_iota == top_row)).astype(jnp.float32)

    pfx = jnp.dot(disp_all, lt)

    cnt = jnp.sum(disp_all, axis=1, keepdims=True)
    alg = jnp.maximum(jnp.floor((cnt + (TB - 1)) / TB) * TB, float(TB))
    offs, ends = [], []
    o = jnp.zeros((1, 1), jnp.float32)
    for e in range(E):
        offs.append(o)
        o = o + alg[e:e + 1, :]
        ends.append(o)
    total = o

    base = jnp.zeros((1, S), jnp.float32)
    for e in range(E):
        base = base + disp_all[e:e + 1, :] * (offs[e] + pfx[e:e + 1, :])
    any_row = jnp.sum(disp_all, axis=0, keepdims=True)
    pos_row = jnp.where(any_row > 0, base, POS_NONE)
    src_row = jnp.where(any_row > 0, base, float(P))
    src_ref[...] = src_row.astype(jnp.int32)

    pos_col = jnp.transpose(pos_row)
    t_col = lax.broadcasted_iota(jnp.int32, (S, 1), 0).astype(jnp.float32)
    perm_tiles = []
    for pt in range(P // TJ):
        p_row = (lax.broadcasted_iota(jnp.int32, (1, TJ), 1) + pt * TJ
                 ).astype(jnp.float32)
        eq = (pos_col == p_row).astype(jnp.float32)
        val = jnp.sum(eq * t_col, axis=0, keepdims=True)
        nonempty = jnp.sum(eq, axis=0, keepdims=True) > 0
        spread = p_row - jnp.where(p_row >= float(S), float(S), 0.0)
        perm_tiles.append(jnp.where(nonempty, val, spread))
    perm_ref[...] = jnp.concatenate(perm_tiles, axis=1).astype(jnp.int32)

    b_iota = lax.broadcasted_iota(jnp.int32, (1, 32), 1).astype(jnp.float32)
    bstart = b_iota * TB
    bexp = jnp.zeros((1, 32), jnp.float32)
    for e in range(E):
        bexp = bexp + (bstart >= ends[e]).astype(jnp.float32)
    bexp = jnp.minimum(bexp, float(E - 1))
    bexp_ref[...] = bexp.astype(jnp.int32)
    bval = (bstart < total) & (b_iota < float(NB))
    bval_ref[...] = bval.astype(jnp.int32)
    prev = jnp.concatenate([jnp.full((1, 1), -1.0), bexp[:, :31]], axis=1)
    first_ref[...] = (bval & (bexp != prev)).astype(jnp.int32)


def _route(x, thr):
    return pl.pallas_call(
        _route_body,
        out_shape=(
            jax.ShapeDtypeStruct((1, P), jnp.int32),
            jax.ShapeDtypeStruct((1, S), jnp.int32),
            jax.ShapeDtypeStruct((1, 32), jnp.int32),
            jax.ShapeDtypeStruct((1, 32), jnp.int32),
            jax.ShapeDtypeStruct((1, 32), jnp.int32),
        ),
    )(x, thr)


def _make_gather(n_rows, n_chunk):
    mesh = plsc.VectorSubcoreMesh(core_axis_name="c", subcore_axis_name="s")
    NC, NS = 2, 16
    n_per = n_rows // (NC * NS)

    def body(table_hbm, idx_hbm, out_hbm, idx_v, rows_v, sem):
        wid = lax.axis_index("s") * NC + lax.axis_index("c")
        base = wid * n_per
        for c in range(n_per // n_chunk):
            off = base + c * n_chunk
            pltpu.sync_copy(idx_hbm.at[pl.ds(off, n_chunk)], idx_v)
            pltpu.async_copy(table_hbm.at[idx_v], rows_v, sem).wait()
            pltpu.sync_copy(rows_v, out_hbm.at[pl.ds(off, n_chunk)])

    return functools.partial(
        pl.kernel,
        mesh=mesh,
        out_type=jax.ShapeDtypeStruct((n_rows, H), jnp.float32),
        scratch_types=[
            pltpu.VMEM((n_chunk,), jnp.int32),
            pltpu.VMEM((n_chunk, H), jnp.float32),
            pltpu.SemaphoreType.DMA,
        ],
    )(body)


def _gather_x(table, idx):
    return _make_gather(P, 64)(table, idx)


def _gather_y(table, idx):
    return _make_gather(S, 64)(table, idx)


def _gffn_body(be_ref, bv_ref, fr_ref, xg_ref, w1_ref, b1_ref, w2_ref,
               b2_ref, o_ref, crow_ref, crow_scr):
    b = pl.program_id(0)

    @pl.when(b == 0)
    def _init():
        crow_scr[...] = jnp.zeros((1, H), jnp.float32)

    @pl.when(fr_ref[b] != 0)
    def _const_step():
        h0 = _gelu(b1_ref[0])
        y0 = lax.dot_general(h0, w2_ref[0], (((1,), (1,)), ((), ())))
        crow_scr[...] = crow_scr[...] + y0 + b2_ref[0]

    @pl.when(bv_ref[b] != 0)
    def _compute():
        xb = xg_ref[...]
        h = lax.dot_general(xb, w1_ref[0], (((1,), (1,)), ((), ())))
        b1v = b1_ref[0]
        h = _gelu(h + b1v) - _gelu(b1v)
        y = lax.dot_general(h, w2_ref[0], (((1,), (1,)), ((), ())))
        o_ref[...] = y

    @pl.when(b == NB)
    def _trash():
        o_ref[...] = jnp.zeros((TB, H), jnp.float32)

    crow_ref[...] = crow_scr[...]


def _gffn(bexp, bval, first, xg, W1, b1, W2, b2):
    grid_spec = pltpu.PrefetchScalarGridSpec(
        num_scalar_prefetch=3,
        grid=(NBTOT,),
        in_specs=[
            pl.BlockSpec(
                (TB, H),
                lambda b, be, bv, fr: (jnp.minimum(b, NB - 1), 0)),
            pl.BlockSpec((1, D, H), lambda b, be, bv, fr: (be[b], 0, 0)),
            pl.BlockSpec((1, 1, D), lambda b, be, bv, fr: (be[b], 0, 0)),
            pl.BlockSpec((1, H, D), lambda b, be, bv, fr: (be[b], 0, 0)),
            pl.BlockSpec((1, 1, H), lambda b, be, bv, fr: (be[b], 0, 0)),
        ],
        out_specs=(
            pl.BlockSpec((TB, H), lambda b, be, bv, fr: (b, 0)),
            pl.BlockSpec((1, H), lambda b, be, bv, fr: (0, 0)),
        ),
        scratch_shapes=[pltpu.VMEM((1, H), jnp.float32)],
    )
    return pl.pallas_call(
        _gffn_body,
        grid_spec=grid_spec,
        out_shape=(
            jax.ShapeDtypeStruct((NBTOT * TB, H), jnp.float32),
            jax.ShapeDtypeStruct((1, H), jnp.float32),
        ),
        compiler_params=pltpu.CompilerParams(
            dimension_semantics=("arbitrary",),
        ),
    )(bexp, bval, first, xg, W1, b1[:, None, :], W2, b2[:, None, :])


def _addc_body(a_ref, c_ref, o_ref):
    o_ref[...] = a_ref[...] + c_ref[...]


def _addc(a, c):
    return pl.pallas_call(
        _addc_body,
        out_shape=jax.ShapeDtypeStruct((S, H), jnp.float32),
    )(a, c)


def kernel(inputs, W1, b1, W2, b2, affinity_threshold):
    x = inputs[0]
    thr = jnp.reshape(affinity_threshold, (1, 1)).astype(jnp.float32)
    perm, src, bexp, bval, first = _route(x, thr)
    xg = _gather_x(x, perm.reshape(P))
    yg, crow = _gffn(bexp.reshape(32), bval.reshape(32), first.reshape(32),
                     xg, W1, b1, W2, b2)
    out = _gather_y(yg, src.reshape(S))
    return _addc(out, crow)[None]
